# Initial kernel scaffold; baseline (speedup 1.0000x reference)
import jax
import jax.numpy as jnp
from jax.experimental import pallas as pl


def kernel(x, edge_index, edge_attr, u, params):
    raise NotImplementedError("write your pallas kernel here")


# speedup vs baseline: 8.7153x; 8.7153x over previous
import functools

import jax
import jax.numpy as jnp
from jax import lax
from jax.experimental import pallas as pl
from jax.experimental.pallas import tpu as pltpu
from jax.experimental.pallas import tpu_sc as plsc

N_NODES = 50000
N_EDGES = 3200000
NC = 2
NS = 16
NW = NC * NS
CH = 128
KCH = 8
TILE = CH * KCH
TOTAL_TILES = N_EDGES // TILE
TPW = (TOTAL_TILES + NW - 1) // NW
NP = 50048
ZR = NP // NS

EROWS = N_EDGES // 8
RB = 1000
TN = 2000
ND = N_NODES // TN

_PREC = lax.Precision.HIGHEST
_SC_PARAMS = pltpu.CompilerParams(use_tc_tiling_on_sc=False)


def _sc_mesh():
    return plsc.VectorSubcoreMesh(
        core_axis_name="c", subcore_axis_name="s", num_cores=NC, num_subcores=NS
    )


def _pass_a_body(x16_h, row_h, col_h, z_h, src_o, dst_o, acc_o,
                 idxr, idxc, vsrc, vdst, acc_sh, sem):
    cid = lax.axis_index("c")
    sid = lax.axis_index("s")
    wid = sid * NC + cid

    pltpu.sync_copy(z_h, acc_sh.at[pl.ds(sid * ZR, ZR)])
    plsc.subcore_barrier()

    @pl.loop(0, TPW)
    def _(t):
        tile = t * NW + wid

        @pl.when(tile < TOTAL_TILES)
        def _():
            pltpu.sync_copy(row_h.at[pl.ds(tile * KCH, KCH)], idxr)
            pltpu.sync_copy(col_h.at[pl.ds(tile * KCH, KCH)], idxc)
            cps = []
            for j in range(KCH):
                cps.append(pltpu.async_copy(
                    x16_h.at[idxr.at[j]], vsrc.at[pl.ds(j * CH, CH)], sem))
                cps.append(pltpu.async_copy(
                    x16_h.at[idxc.at[j]], vdst.at[pl.ds(j * CH, CH)], sem))
            for c in cps:
                c.wait()
            base = tile * TILE
            pltpu.sync_copy(vsrc, src_o.at[pl.ds(base, TILE)])
            pltpu.sync_copy(vdst, dst_o.at[pl.ds(base, TILE)])
            for j in range(KCH):
                pltpu.sync_copy(vsrc.at[pl.ds(j * CH, CH)],
                                acc_sh.at[idxc.at[j]], add=True)

    plsc.subcore_barrier()
    pltpu.sync_copy(acc_sh.at[pl.ds(sid * ZR, ZR)],
                    acc_o.at[cid].at[pl.ds(sid * ZR, ZR)])


def _pass_a(x16, row2d, col2d, zrows):
    f32 = jnp.float32
    k = pl.kernel(
        _pass_a_body,
        out_type=[
            jax.ShapeDtypeStruct((N_EDGES, 16), f32),
            jax.ShapeDtypeStruct((N_EDGES, 16), f32),
            jax.ShapeDtypeStruct((NC, NP, 16), f32),
        ],
        mesh=_sc_mesh(),
        scratch_types=[
            pltpu.VMEM((KCH, CH), jnp.int32),
            pltpu.VMEM((KCH, CH), jnp.int32),
            pltpu.VMEM((TILE, 16), f32),
            pltpu.VMEM((TILE, 16), f32),
            pltpu.VMEM_SHARED((NP, 16), f32),
            pltpu.SemaphoreType.DMA,
        ],
        compiler_params=_SC_PARAMS,
    )
    return k(x16, row2d, col2d, zrows)


def _pass_c_body(e16_h, col_h, z_h, acc_o, idxc, vbuf, acc_sh, sem):
    cid = lax.axis_index("c")
    sid = lax.axis_index("s")
    wid = sid * NC + cid

    pltpu.sync_copy(z_h, acc_sh.at[pl.ds(sid * ZR, ZR)])
    plsc.subcore_barrier()

    @pl.loop(0, TPW)
    def _(t):
        tile = t * NW + wid

        @pl.when(tile < TOTAL_TILES)
        def _():
            pltpu.sync_copy(col_h.at[pl.ds(tile * KCH, KCH)], idxc)
            pltpu.async_copy(e16_h.at[pl.ds(tile * TILE, TILE)], vbuf, sem).wait()
            for j in range(KCH):
                pltpu.sync_copy(vbuf.at[pl.ds(j * CH, CH)],
                                acc_sh.at[idxc.at[j]], add=True)

    plsc.subcore_barrier()
    pltpu.sync_copy(acc_sh.at[pl.ds(sid * ZR, ZR)],
                    acc_o.at[cid].at[pl.ds(sid * ZR, ZR)])


def _pass_c(e16, col2d, zrows):
    k = pl.kernel(
        _pass_c_body,
        out_type=jax.ShapeDtypeStruct((NC, NP, 16), jnp.float32),
        mesh=_sc_mesh(),
        scratch_types=[
            pltpu.VMEM((KCH, CH), jnp.int32),
            pltpu.VMEM((TILE, 16), jnp.float32),
            pltpu.VMEM_SHARED((NP, 16), jnp.float32),
            pltpu.SemaphoreType.DMA,
        ],
        compiler_params=_SC_PARAMS,
    )
    return k(e16, col2d, zrows)


def _edge_mlp_body(u_ref, src_p, dst_p, ea_p, bds, bdd, bde, b1p, w1up,
                   bd2, b2p, out_p):
    c = b1p[...] + u_ref[0, 0] * w1up[...]
    h = (jnp.dot(src_p[...], bds[...], precision=_PREC)
         + jnp.dot(dst_p[...], bdd[...], precision=_PREC)
         + jnp.dot(ea_p[...], bde[...], precision=_PREC)
         + c)
    h = jnp.maximum(h, 0.0)
    out_p[...] = jnp.dot(h, bd2[...], precision=_PREC) + b2p[...]


def _edge_mlp(u11, srcp, dstp, eap, bds, bdd, bde, b1p, w1up, bd2, b2p):
    grid = (EROWS // RB,)
    blk = lambda: pl.BlockSpec((RB, 128), lambda i: (i, 0))
    full = lambda s: pl.BlockSpec(s, lambda i: (0, 0))
    return pl.pallas_call(
        _edge_mlp_body,
        grid=grid,
        in_specs=[
            full((1, 1)), blk(), blk(), blk(),
            full((128, 256)), full((128, 256)), full((128, 256)),
            full((1, 256)), full((1, 256)),
            full((256, 128)), full((1, 128)),
        ],
        out_specs=blk(),
        out_shape=jax.ShapeDtypeStruct((EROWS, 128), jnp.float32),
    )(u11, srcp, dstp, eap, bds, bdd, bde, b1p, w1up, bd2, b2p)


def _node_mlp_body(x16_ref, a0, a1, c0, c1, u_ref,
                   wn1x, wn1a, wn1e, wn1u, bn1, wn2, bn2,
                   wg1u, wg1x, bg1, wg2, bg2,
                   x16o, uo, xsum):
    i = pl.program_id(0)
    x = x16_ref[:, :8]
    sx = a0[:, 0:8] + a1[:, 0:8]
    cnt = a0[:, 8:9] + a1[:, 8:9]
    se = c0[:, 0:8] + c1[:, 0:8]
    denom = jnp.maximum(cnt, 1.0)
    ax = sx / denom
    ae = se / denom
    u = u_ref[0, 0]
    cn = bn1[...] + u * wn1u[...]
    h = (jnp.dot(x, wn1x[...], precision=_PREC)
         + jnp.dot(ax, wn1a[...], precision=_PREC)
         + jnp.dot(ae, wn1e[...], precision=_PREC)
         + cn)
    h = jnp.maximum(h, 0.0)
    xn = jnp.dot(h, wn2[...], precision=_PREC) + bn2[...]

    x16o[...] = jnp.concatenate(
        [xn, jnp.ones((TN, 1), jnp.float32), jnp.zeros((TN, 7), jnp.float32)],
        axis=1)

    @pl.when(i == 0)
    def _():
        xsum[...] = jnp.zeros((1, 8), jnp.float32)

    xsum[...] += jnp.sum(xn, axis=0, keepdims=True)

    @pl.when(i == ND - 1)
    def _():
        mean = xsum[...] * (1.0 / N_NODES)
        g = (u * wg1u[...] + jnp.dot(mean, wg1x[...], precision=_PREC)
             + bg1[...])
        g = jnp.maximum(g, 0.0)
        uo[...] = jnp.dot(g, wg2[...], precision=_PREC) + bg2[...]


def _node_mlp(x16, a0, a1, c0, c1, u11,
              wn1x, wn1a, wn1e, wn1u, bn1, wn2, bn2,
              wg1u, wg1x, bg1, wg2, bg2):
    blk16 = pl.BlockSpec((TN, 16), lambda i: (i, 0))
    full = lambda s: pl.BlockSpec(s, lambda i: (0, 0))
    return pl.pallas_call(
        _node_mlp_body,
        grid=(ND,),
        in_specs=[
            blk16, blk16, blk16, blk16, blk16,
            full((1, 1)),
            full((8, 25)), full((8, 25)), full((8, 25)), full((1, 25)),
            full((1, 25)), full((25, 8)), full((1, 8)),
            full((1, 9)), full((8, 9)), full((1, 9)), full((9, 1)),
            full((1, 1)),
        ],
        out_specs=[
            pl.BlockSpec((TN, 16), lambda i: (i, 0)),
            pl.BlockSpec((1, 1), lambda i: (0, 0)),
        ],
        out_shape=[
            jax.ShapeDtypeStruct((N_NODES, 16), jnp.float32),
            jax.ShapeDtypeStruct((1, 1), jnp.float32),
        ],
        scratch_shapes=[pltpu.VMEM((1, 8), jnp.float32)],
    )(x16, a0, a1, c0, c1, u11,
      wn1x, wn1a, wn1e, wn1u, bn1, wn2, bn2,
      wg1u, wg1x, bg1, wg2, bg2)


def _pad2(w, r, c):
    return jnp.zeros((r, c), jnp.float32).at[: w.shape[0], : w.shape[1]].set(w)


def _edge_weights(blk):
    w1, b1 = blk["e1"]
    w2, b2 = blk["e2"]
    eye8 = jnp.eye(8, dtype=jnp.float32)
    bds = jnp.kron(eye8, _pad2(w1[0:8], 16, 32))
    bdd = jnp.kron(eye8, _pad2(w1[8:16], 16, 32))
    bde = jnp.kron(eye8, _pad2(w1[16:24], 16, 32))
    b1p = jnp.tile(_pad2(b1.reshape(1, 25), 1, 32), (1, 8))
    w1up = jnp.tile(_pad2(w1[24:25], 1, 32), (1, 8))
    bd2 = jnp.kron(eye8, _pad2(w2, 32, 16))
    b2p = jnp.tile(_pad2(b2.reshape(1, 8), 1, 16), (1, 8))
    return bds, bdd, bde, b1p, w1up, bd2, b2p


def _node_weights(blk):
    wn1, bn1 = blk["n1"]
    wn2, bn2 = blk["n2"]
    wg1, bg1 = blk["g1"]
    wg2, bg2 = blk["g2"]
    return (wn1[0:8], wn1[8:16], wn1[16:24], wn1[24:25], bn1.reshape(1, 25),
            wn2, bn2.reshape(1, 8),
            wg1[0:1], wg1[1:9], bg1.reshape(1, 9), wg2, bg2.reshape(1, 1))


def kernel(x, edge_index, edge_attr, u, params):
    f32 = jnp.float32
    row2d = edge_index[0].astype(jnp.int32).reshape(N_EDGES // CH, CH)
    col2d = edge_index[1].astype(jnp.int32).reshape(N_EDGES // CH, CH)
    zrows = jnp.zeros((ZR, 16), f32)
    x16 = jnp.concatenate(
        [x.astype(f32), jnp.ones((N_NODES, 1), f32),
         jnp.zeros((N_NODES, 7), f32)], axis=1)
    eap = jnp.pad(edge_attr.astype(f32), ((0, 0), (0, 8))).reshape(EROWS, 128)
    u11 = u.astype(f32).reshape(1, 1)

    for blk in params:
        bds, bdd, bde, b1p, w1up, bd2, b2p = _edge_weights(blk)
        nw = _node_weights(blk)

        srcr, dstr, accA = _pass_a(x16, row2d, col2d, zrows)
        e16p = _edge_mlp(u11, srcr.reshape(EROWS, 128), dstr.reshape(EROWS, 128),
                         eap, bds, bdd, bde, b1p, w1up, bd2, b2p)
        accC = _pass_c(e16p.reshape(N_EDGES, 16), col2d, zrows)
        x16, u11 = _node_mlp(x16, accA[0, :N_NODES], accA[1, :N_NODES],
                             accC[0, :N_NODES], accC[1, :N_NODES], u11, *nw)
        eap = e16p

    return x16[:, :8]


# speedup vs baseline: 11.5739x; 1.3280x over previous
import functools

import jax
import jax.numpy as jnp
from jax import lax
from jax.experimental import pallas as pl
from jax.experimental.pallas import tpu as pltpu
from jax.experimental.pallas import tpu_sc as plsc

N_NODES = 50000
N_EDGES = 3200000
NC = 2
NS = 16
NW = NC * NS
CH = 128
KCH = 8
TILE = CH * KCH
TOTAL_TILES = N_EDGES // TILE
TPW = (TOTAL_TILES + NW - 1) // NW
NP = 50048
ZR = NP // NS

EROWS = N_EDGES // 8
RB = 1000
TN = 2000
ND = N_NODES // TN

_PREC = lax.Precision.HIGHEST
_SC_PARAMS = pltpu.CompilerParams(use_tc_tiling_on_sc=False)


def _sc_mesh():
    return plsc.VectorSubcoreMesh(
        core_axis_name="c", subcore_axis_name="s", num_cores=NC, num_subcores=NS
    )


def _pass_a_body(x16_h, row_h, col_h, z_h, src_o, dst_o, acc_o,
                 idxr, idxc, vsrc, vdst, acc_sh, sem):
    cid = lax.axis_index("c")
    sid = lax.axis_index("s")
    wid = sid * NC + cid

    pltpu.sync_copy(z_h, acc_sh.at[pl.ds(sid * ZR, ZR)])
    plsc.subcore_barrier()

    @pl.loop(0, TPW)
    def _(t):
        tile = t * NW + wid

        @pl.when(tile < TOTAL_TILES)
        def _():
            pltpu.sync_copy(row_h.at[pl.ds(tile * KCH, KCH)], idxr)
            pltpu.sync_copy(col_h.at[pl.ds(tile * KCH, KCH)], idxc)
            cps = []
            for j in range(KCH):
                cps.append(pltpu.async_copy(
                    x16_h.at[idxr.at[j]], vsrc.at[pl.ds(j * CH, CH)], sem))
                cps.append(pltpu.async_copy(
                    x16_h.at[idxc.at[j]], vdst.at[pl.ds(j * CH, CH)], sem))
            for c in cps:
                c.wait()
            base = tile * TILE
            pltpu.sync_copy(vsrc, src_o.at[pl.ds(base, TILE)])
            pltpu.sync_copy(vdst, dst_o.at[pl.ds(base, TILE)])
            for j in range(KCH):
                pltpu.sync_copy(vsrc.at[pl.ds(j * CH, CH)],
                                acc_sh.at[idxc.at[j]], add=True)

    plsc.subcore_barrier()
    pltpu.sync_copy(acc_sh.at[pl.ds(sid * ZR, ZR)],
                    acc_o.at[cid].at[pl.ds(sid * ZR, ZR)])


def _pass_a(x16, row2d, col2d, zrows):
    f32 = jnp.float32
    k = pl.kernel(
        _pass_a_body,
        out_type=[
            jax.ShapeDtypeStruct((N_EDGES, 16), f32),
            jax.ShapeDtypeStruct((N_EDGES, 16), f32),
            jax.ShapeDtypeStruct((NC, NP, 16), f32),
        ],
        mesh=_sc_mesh(),
        scratch_types=[
            pltpu.VMEM((KCH, CH), jnp.int32),
            pltpu.VMEM((KCH, CH), jnp.int32),
            pltpu.VMEM((TILE, 16), f32),
            pltpu.VMEM((TILE, 16), f32),
            pltpu.VMEM_SHARED((NP, 16), f32),
            pltpu.SemaphoreType.DMA,
        ],
        compiler_params=_SC_PARAMS,
    )
    return k(x16, row2d, col2d, zrows)


def _pass_c_body(e16_h, col_h, z_h, acc_o, idxc, vbuf, acc_sh, sem):
    cid = lax.axis_index("c")
    sid = lax.axis_index("s")
    wid = sid * NC + cid

    pltpu.sync_copy(z_h, acc_sh.at[pl.ds(sid * ZR, ZR)])
    plsc.subcore_barrier()

    @pl.loop(0, TPW)
    def _(t):
        tile = t * NW + wid

        @pl.when(tile < TOTAL_TILES)
        def _():
            pltpu.sync_copy(col_h.at[pl.ds(tile * KCH, KCH)], idxc)
            pltpu.async_copy(e16_h.at[pl.ds(tile * TILE, TILE)], vbuf, sem).wait()
            for j in range(KCH):
                pltpu.sync_copy(vbuf.at[pl.ds(j * CH, CH)],
                                acc_sh.at[idxc.at[j]], add=True)

    plsc.subcore_barrier()
    pltpu.sync_copy(acc_sh.at[pl.ds(sid * ZR, ZR)],
                    acc_o.at[cid].at[pl.ds(sid * ZR, ZR)])


def _pass_c(e16, col2d, zrows):
    k = pl.kernel(
        _pass_c_body,
        out_type=jax.ShapeDtypeStruct((NC, NP, 16), jnp.float32),
        mesh=_sc_mesh(),
        scratch_types=[
            pltpu.VMEM((KCH, CH), jnp.int32),
            pltpu.VMEM((TILE, 16), jnp.float32),
            pltpu.VMEM_SHARED((NP, 16), jnp.float32),
            pltpu.SemaphoreType.DMA,
        ],
        compiler_params=_SC_PARAMS,
    )
    return k(e16, col2d, zrows)


def _dot3(a, wh, wl):
    bf16, f32 = jnp.bfloat16, jnp.float32
    ah = a.astype(bf16)
    al = (a - ah.astype(f32)).astype(bf16)
    return (jnp.dot(ah, wh[...], preferred_element_type=f32)
            + jnp.dot(ah, wl[...], preferred_element_type=f32)
            + jnp.dot(al, wh[...], preferred_element_type=f32))


def _edge_mlp_body(u_ref, src_p, dst_p, ea_p, bdsh, bdsl, bddh, bddl,
                   bdeh, bdel, b1p, w1up, bd2h, bd2l, b2p, out_p):
    c = b1p[...] + u_ref[0, 0] * w1up[...]
    h = (_dot3(src_p[...], bdsh, bdsl)
         + _dot3(dst_p[...], bddh, bddl)
         + _dot3(ea_p[...], bdeh, bdel)
         + c)
    h = jnp.maximum(h, 0.0)
    out_p[...] = _dot3(h, bd2h, bd2l) + b2p[...]


def _edge_mlp(u11, srcp, dstp, eap, bds, bdd, bde, b1p, w1up, bd2, b2p):
    grid = (EROWS // RB,)
    blk = lambda: pl.BlockSpec((RB, 128), lambda i: (i, 0))
    full = lambda s: pl.BlockSpec(s, lambda i: (0, 0))

    def split(w):
        wh = w.astype(jnp.bfloat16)
        wl = (w - wh.astype(jnp.float32)).astype(jnp.bfloat16)
        return wh, wl

    bdsh, bdsl = split(bds)
    bddh, bddl = split(bdd)
    bdeh, bdel = split(bde)
    bd2h, bd2l = split(bd2)
    return pl.pallas_call(
        _edge_mlp_body,
        grid=grid,
        in_specs=[
            full((1, 1)), blk(), blk(), blk(),
            full((128, 256)), full((128, 256)),
            full((128, 256)), full((128, 256)),
            full((128, 256)), full((128, 256)),
            full((1, 256)), full((1, 256)),
            full((256, 128)), full((256, 128)), full((1, 128)),
        ],
        out_specs=blk(),
        out_shape=jax.ShapeDtypeStruct((EROWS, 128), jnp.float32),
    )(u11, srcp, dstp, eap, bdsh, bdsl, bddh, bddl, bdeh, bdel,
      b1p, w1up, bd2h, bd2l, b2p)


def _node_mlp_body(x16_ref, a0, a1, c0, c1, u_ref,
                   wn1x, wn1a, wn1e, wn1u, bn1, wn2, bn2,
                   wg1u, wg1x, bg1, wg2, bg2,
                   x16o, uo, xsum):
    i = pl.program_id(0)
    x = x16_ref[:, :8]
    sx = a0[:, 0:8] + a1[:, 0:8]
    cnt = a0[:, 8:9] + a1[:, 8:9]
    se = c0[:, 0:8] + c1[:, 0:8]
    denom = jnp.maximum(cnt, 1.0)
    ax = sx / denom
    ae = se / denom
    u = u_ref[0, 0]
    cn = bn1[...] + u * wn1u[...]
    h = (jnp.dot(x, wn1x[...], precision=_PREC)
         + jnp.dot(ax, wn1a[...], precision=_PREC)
         + jnp.dot(ae, wn1e[...], precision=_PREC)
         + cn)
    h = jnp.maximum(h, 0.0)
    xn = jnp.dot(h, wn2[...], precision=_PREC) + bn2[...]

    x16o[...] = jnp.concatenate(
        [xn, jnp.ones((TN, 1), jnp.float32), jnp.zeros((TN, 7), jnp.float32)],
        axis=1)

    @pl.when(i == 0)
    def _():
        xsum[...] = jnp.zeros((1, 8), jnp.float32)

    xsum[...] += jnp.sum(xn, axis=0, keepdims=True)

    @pl.when(i == ND - 1)
    def _():
        mean = xsum[...] * (1.0 / N_NODES)
        g = (u * wg1u[...] + jnp.dot(mean, wg1x[...], precision=_PREC)
             + bg1[...])
        g = jnp.maximum(g, 0.0)
        uo[...] = jnp.dot(g, wg2[...], precision=_PREC) + bg2[...]


def _node_mlp(x16, a0, a1, c0, c1, u11,
              wn1x, wn1a, wn1e, wn1u, bn1, wn2, bn2,
              wg1u, wg1x, bg1, wg2, bg2):
    blk16 = pl.BlockSpec((TN, 16), lambda i: (i, 0))
    full = lambda s: pl.BlockSpec(s, lambda i: (0, 0))
    return pl.pallas_call(
        _node_mlp_body,
        grid=(ND,),
        in_specs=[
            blk16, blk16, blk16, blk16, blk16,
            full((1, 1)),
            full((8, 25)), full((8, 25)), full((8, 25)), full((1, 25)),
            full((1, 25)), full((25, 8)), full((1, 8)),
            full((1, 9)), full((8, 9)), full((1, 9)), full((9, 1)),
            full((1, 1)),
        ],
        out_specs=[
            pl.BlockSpec((TN, 16), lambda i: (i, 0)),
            pl.BlockSpec((1, 1), lambda i: (0, 0)),
        ],
        out_shape=[
            jax.ShapeDtypeStruct((N_NODES, 16), jnp.float32),
            jax.ShapeDtypeStruct((1, 1), jnp.float32),
        ],
        scratch_shapes=[pltpu.VMEM((1, 8), jnp.float32)],
    )(x16, a0, a1, c0, c1, u11,
      wn1x, wn1a, wn1e, wn1u, bn1, wn2, bn2,
      wg1u, wg1x, bg1, wg2, bg2)


def _pad2(w, r, c):
    return jnp.zeros((r, c), jnp.float32).at[: w.shape[0], : w.shape[1]].set(w)


def _edge_weights(blk):
    w1, b1 = blk["e1"]
    w2, b2 = blk["e2"]
    eye8 = jnp.eye(8, dtype=jnp.float32)
    bds = jnp.kron(eye8, _pad2(w1[0:8], 16, 32))
    bdd = jnp.kron(eye8, _pad2(w1[8:16], 16, 32))
    bde = jnp.kron(eye8, _pad2(w1[16:24], 16, 32))
    b1p = jnp.tile(_pad2(b1.reshape(1, 25), 1, 32), (1, 8))
    w1up = jnp.tile(_pad2(w1[24:25], 1, 32), (1, 8))
    bd2 = jnp.kron(eye8, _pad2(w2, 32, 16))
    b2p = jnp.tile(_pad2(b2.reshape(1, 8), 1, 16), (1, 8))
    return bds, bdd, bde, b1p, w1up, bd2, b2p


def _node_weights(blk):
    wn1, bn1 = blk["n1"]
    wn2, bn2 = blk["n2"]
    wg1, bg1 = blk["g1"]
    wg2, bg2 = blk["g2"]
    return (wn1[0:8], wn1[8:16], wn1[16:24], wn1[24:25], bn1.reshape(1, 25),
            wn2, bn2.reshape(1, 8),
            wg1[0:1], wg1[1:9], bg1.reshape(1, 9), wg2, bg2.reshape(1, 1))


def kernel(x, edge_index, edge_attr, u, params):
    f32 = jnp.float32
    row2d = edge_index[0].astype(jnp.int32).reshape(N_EDGES // CH, CH)
    col2d = edge_index[1].astype(jnp.int32).reshape(N_EDGES // CH, CH)
    zrows = jnp.zeros((ZR, 16), f32)
    x16 = jnp.concatenate(
        [x.astype(f32), jnp.ones((N_NODES, 1), f32),
         jnp.zeros((N_NODES, 7), f32)], axis=1)
    eap = jnp.pad(edge_attr.astype(f32), ((0, 0), (0, 8))).reshape(EROWS, 128)
    u11 = u.astype(f32).reshape(1, 1)

    for blk in params:
        bds, bdd, bde, b1p, w1up, bd2, b2p = _edge_weights(blk)
        nw = _node_weights(blk)

        srcr, dstr, accA = _pass_a(x16, row2d, col2d, zrows)
        e16p = _edge_mlp(u11, srcr.reshape(EROWS, 128), dstr.reshape(EROWS, 128),
                         eap, bds, bdd, bde, b1p, w1up, bd2, b2p)
        accC = _pass_c(e16p.reshape(N_EDGES, 16), col2d, zrows)
        x16, u11 = _node_mlp(x16, accA[0, :N_NODES], accA[1, :N_NODES],
                             accC[0, :N_NODES], accC[1, :N_NODES], u11, *nw)
        eap = e16p

    return x16[:, :8]
